# baseline (speedup 1.0000x reference)
import functools

import jax
import jax.numpy as jnp
from jax import lax
from jax.experimental import pallas as pl
from jax.experimental.pallas import tpu as pltpu
from jax.experimental.pallas import tpu_sc as plsc

NUM_LAYER = 5
EMB = 256
HID = 512
HALF = 128
N_NODES = 10000
N_PAD = 10240
N_EDGES = 160000
N_GRAPHS = 128
GRAPHLET_SZ = 5
N_GRAPHLETS = 2000
NGL_PAD = N_PAD // GRAPHLET_SZ

NC = 2
NS = 16
NHALF = N_NODES // NC
CHUNK = 128
GRP = 8
EALIGN = GRP * CHUNK // 2
E_CAP = 160768
G_TOT = E_CAP // EALIGN
NCH = 2 * E_CAP // CHUNK
DUMP_ROW = NHALF
SROWS = 10240
ZROWS = SROWS // NS
OROWS = 1000


def _make_sc_msg():
    mesh = plsc.VectorSubcoreMesh(core_axis_name="c", subcore_axis_name="s",
                                  num_cores=NC, num_subcores=NS)

    @functools.partial(
        pl.kernel,
        out_type=jax.ShapeDtypeStruct((2 * N_PAD, HALF), jnp.float32),
        mesh=mesh,
        scratch_types=[
            pltpu.VMEM((2, GRP, CHUNK), jnp.int32),
            pltpu.VMEM((2, GRP, CHUNK), jnp.int32),
            pltpu.VMEM((2, CHUNK, HALF), jnp.float32),
            pltpu.VMEM((16,), jnp.int32),
            pltpu.VMEM_SHARED((SROWS, HALF), jnp.float32),
            pltpu.SemaphoreType.DMA((2,)),
            pltpu.SemaphoreType.DMA((2,)),
            pltpu.SemaphoreType.DMA((2,)),
            pltpu.SemaphoreType.DMA((2,)),
        ],
    )
    def sc_msg(h2, srcb, dstb, g1b, zeros, out, idx_s, idx_d, rows, g1_vm,
               acc, isem_s, isem_d, gsem, ssem):
        c = lax.axis_index("c")
        s = lax.axis_index("s")
        pltpu.sync_copy(g1b.at[0, pl.ds(0, 16)], g1_vm)
        g1 = g1_vm[...][0]
        gbase = jnp.where(c == 1, g1, 0)
        gend = jnp.where(c == 1, G_TOT, g1)
        span = gend - gbase - s
        ngs = jnp.maximum(0, (span + NS - 1) // NS)

        pltpu.sync_copy(zeros, acc.at[pl.ds(s * ZROWS, ZROWS)])

        def stage(k, p):
            row = (gbase + s + NS * k) * GRP
            pltpu.async_copy(srcb.at[pl.ds(row, GRP)], idx_s.at[p],
                             isem_s.at[p])
            pltpu.async_copy(dstb.at[pl.ds(row, GRP)], idx_d.at[p],
                             isem_d.at[p])

        def stage_wait(p):
            pltpu.make_async_copy(srcb.at[pl.ds(0, GRP)], idx_s.at[p],
                                  isem_s.at[p]).wait()
            pltpu.make_async_copy(dstb.at[pl.ds(0, GRP)], idx_d.at[p],
                                  isem_d.at[p]).wait()

        @pl.when(ngs > 0)
        def _():
            stage(0, 0)

        plsc.subcore_barrier()

        def chunk_body(i, carry, p=None):
            jj = i * 2
            g0 = pltpu.async_copy(h2.at[idx_s.at[p, jj]], rows.at[0],
                                  gsem.at[0])
            gx = pltpu.async_copy(h2.at[idx_s.at[p, jj + 1]], rows.at[1],
                                  gsem.at[1])
            g0.wait()
            s0 = pltpu.async_copy(rows.at[0], acc.at[idx_d.at[p, jj]],
                                  ssem.at[0], add=True)
            gx.wait()
            s1 = pltpu.async_copy(rows.at[1], acc.at[idx_d.at[p, jj + 1]],
                                  ssem.at[1], add=True)
            s0.wait()
            s1.wait()
            return carry

        def pair_body(i, carry):
            for p in (0, 1):
                k = 2 * i + p

                @pl.when(k < ngs)
                def _(k=k, p=p):
                    stage_wait(p)

                    @pl.when(k + 1 < ngs)
                    def _():
                        stage(k + 1, 1 - p)

                    lax.fori_loop(0, GRP // 2,
                                  functools.partial(chunk_body, p=p), 0,
                                  unroll=False)

            return carry

        lax.fori_loop(0, (ngs + 1) // 2, pair_body, 0, unroll=False)
        plsc.subcore_barrier()

        @pl.when(s < 2 * NHALF // OROWS)
        def _():
            pltpu.sync_copy(acc.at[pl.ds(s * OROWS, OROWS)],
                            out.at[pl.ds(c * 2 * NHALF + s * OROWS, OROWS)])

    return sc_msg


_sc_msg_cache = []


def _sc_msg(h2, srcb, dstb, g1b, zeros):
    if not _sc_msg_cache:
        _sc_msg_cache.append(_make_sc_msg())
    return _sc_msg_cache[0](h2, srcb, dstb, g1b, zeros)


_MLP_BR = 2048


def _mlp_block(h_ref, m_ref, w1_ref, b1_ref, w2_ref, b2_ref, o_ref, *, last):
    z = h_ref[...] + m_ref[...]
    a = jnp.maximum(jnp.dot(z, w1_ref[...],
                            preferred_element_type=jnp.float32) + b1_ref[...],
                    0.0)
    o = jnp.dot(a, w2_ref[...], preferred_element_type=jnp.float32) + b2_ref[...]
    if not last:
        o = jnp.maximum(o, 0.0)
    o_ref[...] = o


def _make_mlp(last):
    grid = N_PAD // _MLP_BR
    return pl.pallas_call(
        functools.partial(_mlp_block, last=last),
        grid=(grid,),
        in_specs=[
            pl.BlockSpec((_MLP_BR, EMB), lambda i: (i, 0)),
            pl.BlockSpec((_MLP_BR, EMB), lambda i: (i, 0)),
            pl.BlockSpec((EMB, HID), lambda i: (0, 0)),
            pl.BlockSpec((1, HID), lambda i: (0, 0)),
            pl.BlockSpec((HID, EMB), lambda i: (0, 0)),
            pl.BlockSpec((1, EMB), lambda i: (0, 0)),
        ],
        out_specs=pl.BlockSpec((_MLP_BR, EMB), lambda i: (i, 0)),
        out_shape=jax.ShapeDtypeStruct((N_PAD, EMB), jnp.float32),
    )


_mlp_mid = _make_mlp(last=False)
_mlp_last = _make_mlp(last=True)


def _final_block(h_ref, g_ref, o_ref):
    g = g_ref[...]
    norm = g / (jnp.sum(g, axis=1, keepdims=True) + 1e-4)
    r = lax.broadcasted_iota(jnp.int32, (GRAPHLET_SZ * EMB, EMB), 0)
    cidx = lax.broadcasted_iota(jnp.int32, (GRAPHLET_SZ * EMB, EMB), 1)
    K = (r % EMB == cidx).astype(jnp.float32)
    pooled = jnp.dot(h_ref[...], K, preferred_element_type=jnp.float32)
    o_ref[...] = jnp.dot(norm, pooled[:N_GRAPHLETS],
                         preferred_element_type=jnp.float32)


_final = pl.pallas_call(
    _final_block,
    in_specs=[
        pl.BlockSpec((NGL_PAD, GRAPHLET_SZ * EMB), lambda: (0, 0)),
        pl.BlockSpec((N_GRAPHS, N_GRAPHLETS), lambda: (0, 0)),
    ],
    out_specs=pl.BlockSpec((N_GRAPHS, EMB), lambda: (0, 0)),
    out_shape=jax.ShapeDtypeStruct((N_GRAPHS, EMB), jnp.float32),
)


def kernel(x, edge_index, graph_has_graphlet, W1, b1, W2, b2):
    src = edge_index[0].astype(jnp.int32)
    dst = edge_index[1].astype(jnp.int32)

    is1 = dst >= NHALF
    cnt0 = jnp.sum(1 - is1.astype(jnp.int32))
    off1 = ((cnt0 + EALIGN - 1) // EALIGN) * EALIGN
    c0 = jnp.cumsum(1 - is1.astype(jnp.int32))
    c1 = jnp.cumsum(is1.astype(jnp.int32))
    pos = jnp.where(is1, off1 + c1 - 1, c0 - 1)
    srcp = jnp.zeros((E_CAP,), jnp.int32).at[pos].set(src)
    dst_local = dst - jnp.where(is1, NHALF, 0)
    dstp = jnp.full((E_CAP,), DUMP_ROW, jnp.int32).at[pos].set(dst_local)
    src2 = jnp.stack([2 * srcp, 2 * srcp + 1], axis=-1).reshape(NCH, CHUNK)
    dst2 = jnp.stack([2 * dstp, 2 * dstp + 1], axis=-1).reshape(NCH, CHUNK)
    g1b = jnp.zeros((8, 128), jnp.int32).at[0, 0].set(off1 // EALIGN)
    zeros = jnp.zeros((ZROWS, HALF), jnp.float32)

    h = jnp.concatenate([x, jnp.zeros((N_PAD - N_NODES, EMB), jnp.float32)])
    for l in range(NUM_LAYER):
        msg2 = _sc_msg(h.reshape(2 * N_PAD, HALF), src2, dst2, g1b, zeros)
        mlp = _mlp_last if l == NUM_LAYER - 1 else _mlp_mid
        h = mlp(h, msg2.reshape(N_PAD, EMB), W1[l], b1[l].reshape(1, HID),
                W2[l], b2[l].reshape(1, EMB))
    h_r = h.reshape(NGL_PAD, GRAPHLET_SZ * EMB)
    return _final(h_r, graph_has_graphlet)


# speedup vs baseline: 1.2602x; 1.2602x over previous
import functools

import jax
import jax.numpy as jnp
from jax import lax
from jax.experimental import pallas as pl
from jax.experimental.pallas import tpu as pltpu
from jax.experimental.pallas import tpu_sc as plsc

NUM_LAYER = 5
EMB = 256
HID = 512
HALF = 128
N_NODES = 10000
N_EDGES = 160000
N_GRAPHS = 128
GRAPHLET_SZ = 5
N_GRAPHLETS = 2000

NC = 2
NS = 16
CHUNK = 128
E_PAD = 163840
CHUNKS_TOTAL = E_PAD // CHUNK
CHUNKS_PER_TILE = CHUNKS_TOTAL // NS
GRP = 8
NGRP = CHUNKS_PER_TILE // GRP
DUMP_ROW = N_NODES
SROWS = 10240
ZROWS = SROWS // NS
OROWS = 1000


def _make_sc_msg():
    mesh = plsc.VectorSubcoreMesh(core_axis_name="c", subcore_axis_name="s",
                                  num_cores=NC, num_subcores=NS)

    @functools.partial(
        pl.kernel,
        out_type=jax.ShapeDtypeStruct((NC, N_NODES, HALF), jnp.float32),
        mesh=mesh,
        scratch_types=[
            pltpu.VMEM((2, GRP, CHUNK), jnp.int32),
            pltpu.VMEM((2, GRP, CHUNK), jnp.int32),
            pltpu.VMEM((2, CHUNK, HALF), jnp.float32),
            pltpu.VMEM_SHARED((SROWS, HALF), jnp.float32),
            pltpu.SemaphoreType.DMA((2,)),
            pltpu.SemaphoreType.DMA((2,)),
            pltpu.SemaphoreType.DMA((2,)),
            pltpu.SemaphoreType.DMA((2,)),
        ],
    )
    def sc_msg(h2, srcb, dstb, zeros, out, idx_s, idx_d, rows, acc,
               isem_s, isem_d, gsem, ssem):
        c = lax.axis_index("c")
        s = lax.axis_index("s")
        base = s * CHUNKS_PER_TILE
        pltpu.sync_copy(zeros, acc.at[pl.ds(s * ZROWS, ZROWS)])

        def stage(g, p):
            ds = pltpu.async_copy(srcb.at[c, pl.ds(base + g * GRP, GRP)],
                                  idx_s.at[p], isem_s.at[p])
            dd = pltpu.async_copy(dstb.at[pl.ds(base + g * GRP, GRP)],
                                  idx_d.at[p], isem_d.at[p])
            return ds, dd

        idx_pend = stage(0, 0)
        plsc.subcore_barrier()

        for g in range(NGRP):
            p = g % 2
            idx_pend[0].wait()
            idx_pend[1].wait()
            if g + 1 < NGRP:
                idx_pend = stage(g + 1, 1 - p)

            def body(i, carry, p=p):
                jj = i * 2
                g0 = pltpu.async_copy(h2.at[idx_s.at[p, jj]], rows.at[0],
                                      gsem.at[0])
                g1 = pltpu.async_copy(h2.at[idx_s.at[p, jj + 1]], rows.at[1],
                                      gsem.at[1])
                g0.wait()
                s0 = pltpu.async_copy(rows.at[0], acc.at[idx_d.at[p, jj]],
                                      ssem.at[0], add=True)
                g1.wait()
                s1 = pltpu.async_copy(rows.at[1], acc.at[idx_d.at[p, jj + 1]],
                                      ssem.at[1], add=True)
                s0.wait()
                s1.wait()
                return carry

            lax.fori_loop(0, GRP // 2, body, 0, unroll=False)
        plsc.subcore_barrier()

        @pl.when(s < N_NODES // OROWS)
        def _():
            pltpu.sync_copy(acc.at[pl.ds(s * OROWS, OROWS)],
                            out.at[c, pl.ds(s * OROWS, OROWS)])

    return sc_msg


_sc_msg_cache = []


def _sc_msg(h2, srcb, dstb, zeros):
    if not _sc_msg_cache:
        _sc_msg_cache.append(_make_sc_msg())
    return _sc_msg_cache[0](h2, srcb, dstb, zeros)


_MLP_BR = 2000


def _mlp_block(h_ref, m_ref, w1_ref, b1_ref, w2_ref, b2_ref, o_ref, *, last):
    h = jnp.concatenate([h_ref[0], h_ref[1]], axis=-1)
    m = jnp.concatenate([m_ref[0], m_ref[1]], axis=-1)
    z = (h + m).astype(jnp.bfloat16)
    a = jnp.maximum(jnp.dot(z, w1_ref[...],
                            preferred_element_type=jnp.float32) + b1_ref[...],
                    0.0)
    o = jnp.dot(a.astype(jnp.bfloat16), w2_ref[...],
                preferred_element_type=jnp.float32) + b2_ref[...]
    if not last:
        o = jnp.maximum(o, 0.0)
    o_ref[0] = o[:, :HALF]
    o_ref[1] = o[:, HALF:]


def _make_mlp(last):
    grid = N_NODES // _MLP_BR
    return pl.pallas_call(
        functools.partial(_mlp_block, last=last),
        grid=(grid,),
        in_specs=[
            pl.BlockSpec((NC, _MLP_BR, HALF), lambda i: (0, i, 0)),
            pl.BlockSpec((NC, _MLP_BR, HALF), lambda i: (0, i, 0)),
            pl.BlockSpec((EMB, HID), lambda i: (0, 0)),
            pl.BlockSpec((1, HID), lambda i: (0, 0)),
            pl.BlockSpec((HID, EMB), lambda i: (0, 0)),
            pl.BlockSpec((1, EMB), lambda i: (0, 0)),
        ],
        out_specs=pl.BlockSpec((NC, _MLP_BR, HALF), lambda i: (0, i, 0)),
        out_shape=jax.ShapeDtypeStruct((NC, N_NODES, HALF), jnp.float32),
    )


_mlp_mid = _make_mlp(last=False)
_mlp_last = _make_mlp(last=True)


def _final_block(h_ref, g_ref, o_ref):
    g = g_ref[...]
    norm = g / (jnp.sum(g, axis=1, keepdims=True) + 1e-4)
    r = lax.broadcasted_iota(jnp.int32, (GRAPHLET_SZ * HALF, HALF), 0)
    cidx = lax.broadcasted_iota(jnp.int32, (GRAPHLET_SZ * HALF, HALF), 1)
    K = (r % HALF == cidx).astype(jnp.float32)
    p0 = jnp.dot(h_ref[0], K, preferred_element_type=jnp.float32)
    p1 = jnp.dot(h_ref[1], K, preferred_element_type=jnp.float32)
    o_ref[:, :HALF] = jnp.dot(norm, p0, preferred_element_type=jnp.float32)
    o_ref[:, HALF:] = jnp.dot(norm, p1, preferred_element_type=jnp.float32)


_final = pl.pallas_call(
    _final_block,
    in_specs=[
        pl.BlockSpec((NC, N_GRAPHLETS, GRAPHLET_SZ * HALF),
                     lambda: (0, 0, 0)),
        pl.BlockSpec((N_GRAPHS, N_GRAPHLETS), lambda: (0, 0)),
    ],
    out_specs=pl.BlockSpec((N_GRAPHS, EMB), lambda: (0, 0)),
    out_shape=jax.ShapeDtypeStruct((N_GRAPHS, EMB), jnp.float32),
)


def kernel(x, edge_index, graph_has_graphlet, W1, b1, W2, b2):
    src = edge_index[0].astype(jnp.int32)
    dst = edge_index[1].astype(jnp.int32)
    src_p = jnp.concatenate([src, jnp.zeros((E_PAD - N_EDGES,), jnp.int32)])
    dst_p = jnp.concatenate(
        [dst, jnp.full((E_PAD - N_EDGES,), DUMP_ROW, jnp.int32)])
    src2 = src_p.reshape(CHUNKS_TOTAL, CHUNK)
    srcb = jnp.stack([src2, src2 + N_NODES])
    dstb = dst_p.reshape(CHUNKS_TOTAL, CHUNK)
    zeros = jnp.zeros((ZROWS, HALF), jnp.float32)

    h2 = x.reshape(N_NODES, NC, HALF).transpose(1, 0, 2)
    for l in range(NUM_LAYER):
        msg2 = _sc_msg(h2.reshape(NC * N_NODES, HALF), srcb, dstb, zeros)
        mlp = _mlp_last if l == NUM_LAYER - 1 else _mlp_mid
        h2 = mlp(h2, msg2, W1[l].astype(jnp.bfloat16), b1[l].reshape(1, HID),
                 W2[l].astype(jnp.bfloat16), b2[l].reshape(1, EMB))
    h_r = h2.reshape(NC, N_GRAPHLETS, GRAPHLET_SZ * HALF)
    return _final(h_r, graph_has_graphlet)


# speedup vs baseline: 1.3503x; 1.0715x over previous
import functools

import jax
import jax.numpy as jnp
from jax import lax
from jax.experimental import pallas as pl
from jax.experimental.pallas import tpu as pltpu
from jax.experimental.pallas import tpu_sc as plsc

NUM_LAYER = 5
EMB = 256
HID = 512
HALF = 128
N_NODES = 10000
N_EDGES = 160000
N_GRAPHS = 128
GRAPHLET_SZ = 5
N_GRAPHLETS = 2000

NC = 2
NS = 16
CHUNK = 128
E_PAD = 163840
CHUNKS_TOTAL = E_PAD // CHUNK
CHUNKS_PER_TILE = CHUNKS_TOTAL // NS
GRP = 8
NGRP = CHUNKS_PER_TILE // GRP
DUMP_ROW = N_NODES
SROWS = 10240
ZROWS = SROWS // NS
OROWS = 1000


def _make_sc_msg():
    mesh = plsc.VectorSubcoreMesh(core_axis_name="c", subcore_axis_name="s",
                                  num_cores=NC, num_subcores=NS)

    @functools.partial(
        pl.kernel,
        out_type=jax.ShapeDtypeStruct((NC, N_NODES, HALF), jnp.float32),
        mesh=mesh,
        scratch_types=[
            pltpu.VMEM((2, GRP, CHUNK), jnp.int32),
            pltpu.VMEM((2, GRP, CHUNK), jnp.int32),
            pltpu.VMEM((2, CHUNK, HALF), jnp.float32),
            pltpu.VMEM_SHARED((SROWS, HALF), jnp.float32),
            pltpu.SemaphoreType.DMA((2,)),
            pltpu.SemaphoreType.DMA((2,)),
            pltpu.SemaphoreType.DMA((2,)),
            pltpu.SemaphoreType.DMA((2,)),
        ],
    )
    def sc_msg(h2, srcb, dstb, zeros, out, idx_s, idx_d, rows, acc,
               isem_s, isem_d, gsem, ssem):
        c = lax.axis_index("c")
        s = lax.axis_index("s")
        base = s * CHUNKS_PER_TILE
        pltpu.sync_copy(zeros, acc.at[pl.ds(s * ZROWS, ZROWS)])

        def stage(g, p):
            ds = pltpu.async_copy(srcb.at[c, pl.ds(base + g * GRP, GRP)],
                                  idx_s.at[p], isem_s.at[p])
            dd = pltpu.async_copy(dstb.at[pl.ds(base + g * GRP, GRP)],
                                  idx_d.at[p], isem_d.at[p])
            return ds, dd

        idx_pend = stage(0, 0)
        plsc.subcore_barrier()

        for g in range(NGRP):
            p = g % 2
            idx_pend[0].wait()
            idx_pend[1].wait()
            if g + 1 < NGRP:
                idx_pend = stage(g + 1, 1 - p)

            def body(i, carry, p=p):
                jj = i * 2
                g0 = pltpu.async_copy(h2.at[idx_s.at[p, jj]], rows.at[0],
                                      gsem.at[0])
                g1 = pltpu.async_copy(h2.at[idx_s.at[p, jj + 1]], rows.at[1],
                                      gsem.at[1])
                g0.wait()
                s0 = pltpu.async_copy(rows.at[0], acc.at[idx_d.at[p, jj]],
                                      ssem.at[0], add=True)
                g1.wait()
                s1 = pltpu.async_copy(rows.at[1], acc.at[idx_d.at[p, jj + 1]],
                                      ssem.at[1], add=True)
                s0.wait()
                s1.wait()
                return carry

            lax.fori_loop(0, GRP // 2, body, 0, unroll=False)
        plsc.subcore_barrier()

        @pl.when(s < N_NODES // OROWS)
        def _():
            pltpu.sync_copy(acc.at[pl.ds(s * OROWS, OROWS)],
                            out.at[c, pl.ds(s * OROWS, OROWS)])

    return sc_msg


_sc_msg_cache = []


def _sc_msg(h2, srcb, dstb, zeros):
    if not _sc_msg_cache:
        _sc_msg_cache.append(_make_sc_msg())
    return _sc_msg_cache[0](h2, srcb, dstb, zeros)


_MLP_BR = 2000


def _mlp_block(h_ref, m_ref, w1_ref, b1_ref, w2_ref, b2_ref, o_ref, *, last):
    h = jnp.concatenate([h_ref[0], h_ref[1]], axis=-1)
    m = jnp.concatenate([m_ref[0], m_ref[1]], axis=-1)
    z = h + m
    a = jnp.maximum(jnp.dot(z, w1_ref[...],
                            preferred_element_type=jnp.float32) + b1_ref[...],
                    0.0)
    o = jnp.dot(a, w2_ref[...], preferred_element_type=jnp.float32) + b2_ref[...]
    if not last:
        o = jnp.maximum(o, 0.0)
    o_ref[0] = o[:, :HALF]
    o_ref[1] = o[:, HALF:]


def _make_mlp(last):
    grid = N_NODES // _MLP_BR
    return pl.pallas_call(
        functools.partial(_mlp_block, last=last),
        grid=(grid,),
        in_specs=[
            pl.BlockSpec((NC, _MLP_BR, HALF), lambda i: (0, i, 0)),
            pl.BlockSpec((NC, _MLP_BR, HALF), lambda i: (0, i, 0)),
            pl.BlockSpec((EMB, HID), lambda i: (0, 0)),
            pl.BlockSpec((1, HID), lambda i: (0, 0)),
            pl.BlockSpec((HID, EMB), lambda i: (0, 0)),
            pl.BlockSpec((1, EMB), lambda i: (0, 0)),
        ],
        out_specs=pl.BlockSpec((NC, _MLP_BR, HALF), lambda i: (0, i, 0)),
        out_shape=jax.ShapeDtypeStruct((NC, N_NODES, HALF), jnp.float32),
    )


_mlp_mid = _make_mlp(last=False)
_mlp_last = _make_mlp(last=True)


def _final_block(h_ref, g_ref, o_ref):
    g = g_ref[...]
    norm = g / (jnp.sum(g, axis=1, keepdims=True) + 1e-4)
    r = lax.broadcasted_iota(jnp.int32, (GRAPHLET_SZ * HALF, HALF), 0)
    cidx = lax.broadcasted_iota(jnp.int32, (GRAPHLET_SZ * HALF, HALF), 1)
    K = (r % HALF == cidx).astype(jnp.float32)
    p0 = jnp.dot(h_ref[0], K, preferred_element_type=jnp.float32)
    p1 = jnp.dot(h_ref[1], K, preferred_element_type=jnp.float32)
    o_ref[:, :HALF] = jnp.dot(norm, p0, preferred_element_type=jnp.float32)
    o_ref[:, HALF:] = jnp.dot(norm, p1, preferred_element_type=jnp.float32)


_final = pl.pallas_call(
    _final_block,
    in_specs=[
        pl.BlockSpec((NC, N_GRAPHLETS, GRAPHLET_SZ * HALF),
                     lambda: (0, 0, 0)),
        pl.BlockSpec((N_GRAPHS, N_GRAPHLETS), lambda: (0, 0)),
    ],
    out_specs=pl.BlockSpec((N_GRAPHS, EMB), lambda: (0, 0)),
    out_shape=jax.ShapeDtypeStruct((N_GRAPHS, EMB), jnp.float32),
)


def kernel(x, edge_index, graph_has_graphlet, W1, b1, W2, b2):
    src = edge_index[0].astype(jnp.int32)
    dst = edge_index[1].astype(jnp.int32)
    src_p = jnp.concatenate([src, jnp.zeros((E_PAD - N_EDGES,), jnp.int32)])
    dst_p = jnp.concatenate(
        [dst, jnp.full((E_PAD - N_EDGES,), DUMP_ROW, jnp.int32)])
    src2 = src_p.reshape(CHUNKS_TOTAL, CHUNK)
    srcb = jnp.stack([src2, src2 + N_NODES])
    dstb = dst_p.reshape(CHUNKS_TOTAL, CHUNK)
    zeros = jnp.zeros((ZROWS, HALF), jnp.float32)

    h2 = x.reshape(N_NODES, NC, HALF).transpose(1, 0, 2)
    for l in range(NUM_LAYER):
        msg2 = _sc_msg(h2.reshape(NC * N_NODES, HALF), srcb, dstb, zeros)
        mlp = _mlp_last if l == NUM_LAYER - 1 else _mlp_mid
        h2 = mlp(h2, msg2, W1[l], b1[l].reshape(1, HID),
                 W2[l], b2[l].reshape(1, EMB))
    h_r = h2.reshape(NC, N_GRAPHLETS, GRAPHLET_SZ * HALF)
    return _final(h_r, graph_has_graphlet)
